# Initial kernel scaffold; baseline (speedup 1.0000x reference)
import jax
import jax.numpy as jnp
from jax.experimental import pallas as pl


def kernel(x_rec, x):
    raise NotImplementedError("write your pallas kernel here")


# speedup vs baseline: 1.3184x; 1.3184x over previous
import jax
import jax.numpy as jnp
from jax.experimental import pallas as pl
from jax.experimental.pallas import tpu as pltpu
import functools

_BLOCK_ROWS = 512


def _loss_kernel(xr_ref, x_ref, num_ref, cnt_ref):
    step = pl.program_id(0)

    @pl.when(step == 0)
    def _init():
        num_ref[0, 0] = 0.0
        cnt_ref[0, 0] = 0.0

    x = x_ref[...]
    xr = xr_ref[...]
    row_sum = jnp.sum(x, axis=1)
    mask = (row_sum != 0).astype(jnp.float32)
    absdiff_row = jnp.sum(jnp.abs(xr - x), axis=1)
    num_ref[0, 0] += jnp.sum(absdiff_row * mask)
    cnt_ref[0, 0] += jnp.sum(mask)


def kernel(x_rec, x):
    margin = 0.5
    B, L, D = x.shape
    rows = B * L
    xr2 = x_rec.reshape(rows, D)
    x2 = x.reshape(rows, D)
    grid = rows // _BLOCK_ROWS

    num, cnt = pl.pallas_call(
        _loss_kernel,
        grid=(grid,),
        in_specs=[
            pl.BlockSpec((_BLOCK_ROWS, D), lambda i: (i, 0)),
            pl.BlockSpec((_BLOCK_ROWS, D), lambda i: (i, 0)),
        ],
        out_specs=[
            pl.BlockSpec((1, 1), lambda i: (0, 0), memory_space=pltpu.SMEM),
            pl.BlockSpec((1, 1), lambda i: (0, 0), memory_space=pltpu.SMEM),
        ],
        out_shape=[
            jax.ShapeDtypeStruct((1, 1), jnp.float32),
            jax.ShapeDtypeStruct((1, 1), jnp.float32),
        ],
        compiler_params=pltpu.CompilerParams(
            dimension_semantics=("arbitrary",),
        ),
    )(xr2, x2)

    return num[0, 0] / (cnt[0, 0] * D) + margin


# speedup vs baseline: 1.5468x; 1.1732x over previous
import jax
import jax.numpy as jnp
from jax.experimental import pallas as pl
from jax.experimental.pallas import tpu as pltpu
import functools

_BLOCK_ROWS = 1024


def _loss_kernel(xr_ref, x_ref, num_ref, cnt_ref):
    step = pl.program_id(0)

    @pl.when(step == 0)
    def _init():
        num_ref[0, 0] = 0.0
        cnt_ref[0, 0] = 0.0

    x = x_ref[...]
    xr = xr_ref[...]
    row_sum = jnp.sum(x, axis=1)
    mask = (row_sum != 0).astype(jnp.float32)
    absdiff_row = jnp.sum(jnp.abs(xr - x), axis=1)
    num_ref[0, 0] += jnp.sum(absdiff_row * mask)
    cnt_ref[0, 0] += jnp.sum(mask)


def kernel(x_rec, x):
    margin = 0.5
    B, L, D = x.shape
    rows = B * L
    xr2 = x_rec.reshape(rows, D)
    x2 = x.reshape(rows, D)
    grid = rows // _BLOCK_ROWS

    num, cnt = pl.pallas_call(
        _loss_kernel,
        grid=(grid,),
        in_specs=[
            pl.BlockSpec((_BLOCK_ROWS, D), lambda i: (i, 0)),
            pl.BlockSpec((_BLOCK_ROWS, D), lambda i: (i, 0)),
        ],
        out_specs=[
            pl.BlockSpec((1, 1), lambda i: (0, 0), memory_space=pltpu.SMEM),
            pl.BlockSpec((1, 1), lambda i: (0, 0), memory_space=pltpu.SMEM),
        ],
        out_shape=[
            jax.ShapeDtypeStruct((1, 1), jnp.float32),
            jax.ShapeDtypeStruct((1, 1), jnp.float32),
        ],
        compiler_params=pltpu.CompilerParams(
            dimension_semantics=("arbitrary",),
        ),
    )(xr2, x2)

    return num[0, 0] / (cnt[0, 0] * D) + margin
